# Initial kernel scaffold; baseline (speedup 1.0000x reference)
import jax
import jax.numpy as jnp
from jax import lax
from jax.experimental import pallas as pl
from jax.experimental.pallas import tpu as pltpu
from jax.experimental.pallas import tpu_sc as plsc

N_USER = 20000
N_ITEM = 30000
N = N_USER + N_ITEM
E = 800000
D = 64
L = 3
H = D // 2

NUM_TILES = 16
CHUNK = 1024
SUB = 128
CHUNKS_PER_TILE = 49
EDGES_PER_TILE = CHUNK * CHUNKS_PER_TILE
E_PAD = EDGES_PER_TILE * NUM_TILES
NODES_PER_TILE = N // NUM_TILES
WB = 625


def _spmm_body(table2, src, dst2, w, out, src_v, gidx_v, w_v, dst_v, rows, sem):
    h = jnp.int32(lax.axis_index("c"))
    s = jnp.int32(lax.axis_index("s"))

    def body(acc):
        zero16 = jnp.zeros((16,), jnp.float32)

        @pl.loop(0, WB, unroll=8)
        def _zero(i):
            rows[i, pl.ds(0, 16)] = zero16
            rows[i, pl.ds(16, 16)] = zero16

        r0 = s * NODES_PER_TILE
        for k in range(NODES_PER_TILE // WB):
            pltpu.sync_copy(rows.at[pl.ds(0, WB)], acc.at[pl.ds(r0 + k * WB, WB)])
        plsc.subcore_barrier()

        e0 = s * EDGES_PER_TILE

        @pl.loop(0, CHUNKS_PER_TILE)
        def _chunk(ci):
            eb = e0 + ci * CHUNK
            eb_sub = s * (EDGES_PER_TILE // SUB) + ci * (CHUNK // SUB)
            pltpu.sync_copy(src.at[pl.ds(eb, CHUNK)], src_v)
            pltpu.sync_copy(w.at[pl.ds(eb, CHUNK)], w_v)
            pltpu.sync_copy(dst2.at[pl.ds(eb_sub, CHUNK // SUB)], dst_v)

            @pl.loop(0, CHUNK // 16, unroll=4)
            def _gidx(i):
                sv = src_v[pl.ds(i * 16, 16)]
                gidx_v[pl.ds(i * 16, 16)] = sv * 2 + h

            descs = [
                pltpu.async_copy(
                    table2.at[gidx_v.at[pl.ds(j * SUB, SUB)]],
                    rows.at[pl.ds(j * SUB, SUB)],
                    sem,
                )
                for j in range(CHUNK // SUB)
            ]
            for d in descs:
                d.wait()

            @pl.loop(0, CHUNK, unroll=4)
            def _scale(e):
                wv = jnp.broadcast_to(w_v[e], (16,))
                rows[e, pl.ds(0, 16)] = rows[e, pl.ds(0, 16)] * wv
                rows[e, pl.ds(16, 16)] = rows[e, pl.ds(16, 16)] * wv

            for j in range(CHUNK // SUB):
                pltpu.sync_copy(
                    rows.at[pl.ds(j * SUB, SUB)],
                    acc.at[dst_v.at[j]],
                    add=True,
                )

        plsc.subcore_barrier()

        for k in range(NODES_PER_TILE // WB):
            pltpu.sync_copy(acc.at[pl.ds(r0 + k * WB, WB)], rows.at[pl.ds(0, WB)])
            pltpu.sync_copy(rows.at[pl.ds(0, WB)], out.at[h, pl.ds(r0 + k * WB, WB)])

    pl.run_scoped(body, acc=pltpu.VMEM_SHARED((N, H), jnp.float32))


_spmm = pl.kernel(
    _spmm_body,
    out_type=jax.ShapeDtypeStruct((2, N, H), jnp.float32),
    mesh=plsc.VectorSubcoreMesh(core_axis_name="c", subcore_axis_name="s"),
    scratch_types=[
        pltpu.VMEM((CHUNK,), jnp.int32),
        pltpu.VMEM((CHUNK,), jnp.int32),
        pltpu.VMEM((CHUNK,), jnp.float32),
        pltpu.VMEM((CHUNK // SUB, SUB), jnp.int32),
        pltpu.VMEM((CHUNK, H), jnp.float32),
        pltpu.SemaphoreType.DMA,
    ],
)


BN = 1000


def _dense_body(a0, a1, el, wf, bf, wb, bb, enew_ref, norm_ref):
    agg = jnp.concatenate([a0[0], a1[0]], axis=1)
    el_v = el[...]
    front = agg + el_v
    fc = front @ wf[...] + bf[...]
    fc = jnp.where(fc >= 0, fc, 0.01 * fc)
    back = (el_v * front) @ wb[...] + bb[...]
    back = jnp.where(back >= 0, back, 0.01 * back)
    enew = fc + back
    nrm = jnp.sqrt(jnp.sum(enew * enew, axis=1, keepdims=True))
    norm_ref[...] = enew / jnp.maximum(nrm, 1e-12)
    enew_ref[...] = enew


_dense = pl.pallas_call(
    _dense_body,
    grid=(N // BN,),
    in_specs=[
        pl.BlockSpec((1, BN, H), lambda i: (0, i, 0)),
        pl.BlockSpec((1, BN, H), lambda i: (1, i, 0)),
        pl.BlockSpec((BN, D), lambda i: (i, 0)),
        pl.BlockSpec((D, D), lambda i: (0, 0)),
        pl.BlockSpec((1, D), lambda i: (0, 0)),
        pl.BlockSpec((D, D), lambda i: (0, 0)),
        pl.BlockSpec((1, D), lambda i: (0, 0)),
    ],
    out_specs=[
        pl.BlockSpec((BN, D), lambda i: (i, 0)),
        pl.BlockSpec((BN, D), lambda i: (i, 0)),
    ],
    out_shape=[
        jax.ShapeDtypeStruct((N, D), jnp.float32),
        jax.ShapeDtypeStruct((N, D), jnp.float32),
    ],
    compiler_params=pltpu.CompilerParams(
        dimension_semantics=("arbitrary",),
    ),
)


def kernel(H_edge_index, H_edge_weight, user_emb, item_emb, Wf, bf, Wb, bb):
    E_l = jnp.concatenate([user_emb, item_emb], axis=0)
    src = H_edge_index[0].astype(jnp.int32)
    dst = H_edge_index[1].astype(jnp.int32)
    w = H_edge_weight.astype(jnp.float32)

    pad = E_PAD - E
    src_p = jnp.pad(src, (0, pad))
    dst_p = jnp.pad(dst, (0, pad)).reshape(E_PAD // SUB, SUB)
    w_p = jnp.pad(w, (0, pad))

    outs = [E_l]
    for i in range(L):
        table2 = E_l.reshape(2 * N, H)
        agg2 = _spmm(table2, src_p, dst_p, w_p)
        E_l, nrm = _dense(
            agg2, agg2, E_l,
            Wf[i], bf[i].reshape(1, D), Wb[i], bb[i].reshape(1, D),
        )
        outs.append(nrm)

    all_emb = jnp.concatenate(outs, axis=1)
    return all_emb[:N_USER], all_emb[N_USER:]


# speedup vs baseline: 5.2877x; 5.2877x over previous
import jax
import jax.numpy as jnp
from jax import lax
from jax.experimental import pallas as pl
from jax.experimental.pallas import tpu as pltpu
from jax.experimental.pallas import tpu_sc as plsc

N_USER = 20000
N_ITEM = 30000
N = N_USER + N_ITEM
E = 800000
D = 64
L = 3
H = D // 2

NUM_TILES = 16
CHUNK = 512
SUB = 128
CHUNKS_PER_TILE = 98
EDGES_PER_TILE = CHUNK * CHUNKS_PER_TILE
E_PAD = EDGES_PER_TILE * NUM_TILES
N_PAD = 50176
NODES_PER_TILE = N_PAD // NUM_TILES
WB = 392


def _spmm_body(table2, src, dst2, w, out, src_v, gidx_v, w_v, dst_v, rows, acc, sem):
    h = jnp.int32(lax.axis_index("c"))
    s = jnp.int32(lax.axis_index("s"))

    if True:
        zero16 = jnp.zeros((16,), jnp.float32)

        @pl.loop(0, WB, unroll=8)
        def _zero(i):
            rows[i, pl.ds(0, 16)] = zero16
            rows[i, pl.ds(16, 16)] = zero16

        r0 = s * NODES_PER_TILE
        for k in range(NODES_PER_TILE // WB):
            pltpu.sync_copy(rows.at[pl.ds(0, WB)], acc.at[pl.ds(r0 + k * WB, WB)])
        plsc.subcore_barrier()

        e0 = s * EDGES_PER_TILE

        @pl.loop(0, CHUNKS_PER_TILE)
        def _chunk(ci):
            eb = e0 + ci * CHUNK
            eb_sub = s * (EDGES_PER_TILE // SUB) + ci * (CHUNK // SUB)
            pltpu.sync_copy(src.at[pl.ds(eb, CHUNK)], src_v)
            pltpu.sync_copy(w.at[pl.ds(eb, CHUNK)], w_v)
            pltpu.sync_copy(dst2.at[pl.ds(eb_sub, CHUNK // SUB)], dst_v)

            @pl.loop(0, CHUNK // 16, unroll=4)
            def _gidx(i):
                sv = src_v[pl.ds(i * 16, 16)]
                gidx_v[pl.ds(i * 16, 16)] = sv * 2 + h

            descs = [
                pltpu.async_copy(
                    table2.at[gidx_v.at[pl.ds(j * SUB, SUB)]],
                    rows.at[pl.ds(j * SUB, SUB)],
                    sem,
                )
                for j in range(CHUNK // SUB)
            ]
            for d in descs:
                d.wait()

            @pl.loop(0, CHUNK // 16)
            def _scale(g):
                wvec = w_v[pl.ds(g * 16, 16)]
                for l in range(16):
                    e = g * 16 + l
                    wv = jnp.broadcast_to(wvec[l], (16,))
                    rows[e, pl.ds(0, 16)] = rows[e, pl.ds(0, 16)] * wv
                    rows[e, pl.ds(16, 16)] = rows[e, pl.ds(16, 16)] * wv

            for j in range(CHUNK // SUB):
                pltpu.sync_copy(
                    rows.at[pl.ds(j * SUB, SUB)],
                    acc.at[dst_v.at[j]],
                    add=True,
                )

        plsc.subcore_barrier()

        for k in range(NODES_PER_TILE // WB):
            pltpu.sync_copy(acc.at[pl.ds(r0 + k * WB, WB)], rows.at[pl.ds(0, WB)])
            pltpu.sync_copy(rows.at[pl.ds(0, WB)], out.at[h, pl.ds(r0 + k * WB, WB)])


_spmm = pl.kernel(
    _spmm_body,
    out_type=jax.ShapeDtypeStruct((2, N_PAD, H), jnp.float32),
    mesh=plsc.VectorSubcoreMesh(core_axis_name="c", subcore_axis_name="s"),
    scratch_types=[
        pltpu.VMEM((CHUNK,), jnp.int32),
        pltpu.VMEM((CHUNK,), jnp.int32),
        pltpu.VMEM((CHUNK,), jnp.float32),
        pltpu.VMEM((CHUNK // SUB, SUB), jnp.int32),
        pltpu.VMEM((CHUNK, H), jnp.float32),
        pltpu.VMEM_SHARED((N_PAD, H), jnp.float32),
        pltpu.SemaphoreType.DMA,
    ],
    compiler_params=pltpu.CompilerParams(use_tc_tiling_on_sc=False),
)


BN = 1000


def _dense_body(a0, a1, el, wf, bf, wb, bb, enew_ref, norm_ref):
    agg = jnp.concatenate([a0[0], a1[0]], axis=1)
    el_v = el[...]
    front = agg + el_v
    fc = front @ wf[...] + bf[...]
    fc = jnp.where(fc >= 0, fc, 0.01 * fc)
    back = (el_v * front) @ wb[...] + bb[...]
    back = jnp.where(back >= 0, back, 0.01 * back)
    enew = fc + back
    nrm = jnp.sqrt(jnp.sum(enew * enew, axis=1, keepdims=True))
    norm_ref[...] = enew / jnp.maximum(nrm, 1e-12)
    enew_ref[...] = enew


_dense = pl.pallas_call(
    _dense_body,
    grid=(N // BN,),
    in_specs=[
        pl.BlockSpec((1, BN, H), lambda i: (0, i, 0)),
        pl.BlockSpec((1, BN, H), lambda i: (1, i, 0)),
        pl.BlockSpec((BN, D), lambda i: (i, 0)),
        pl.BlockSpec((D, D), lambda i: (0, 0)),
        pl.BlockSpec((1, D), lambda i: (0, 0)),
        pl.BlockSpec((D, D), lambda i: (0, 0)),
        pl.BlockSpec((1, D), lambda i: (0, 0)),
    ],
    out_specs=[
        pl.BlockSpec((BN, D), lambda i: (i, 0)),
        pl.BlockSpec((BN, D), lambda i: (i, 0)),
    ],
    out_shape=[
        jax.ShapeDtypeStruct((N, D), jnp.float32),
        jax.ShapeDtypeStruct((N, D), jnp.float32),
    ],
    compiler_params=pltpu.CompilerParams(
        dimension_semantics=("arbitrary",),
    ),
)


def kernel(H_edge_index, H_edge_weight, user_emb, item_emb, Wf, bf, Wb, bb):
    E_l = jnp.concatenate([user_emb, item_emb], axis=0)
    src = H_edge_index[0].astype(jnp.int32)
    dst = H_edge_index[1].astype(jnp.int32)
    w = H_edge_weight.astype(jnp.float32)

    pad = E_PAD - E
    src_p = jnp.pad(src, (0, pad))
    dst_p = jnp.pad(dst, (0, pad)).reshape(E_PAD // SUB, SUB)
    w_p = jnp.pad(w, (0, pad))

    outs = [E_l]
    for i in range(L):
        table2 = E_l.reshape(2 * N, H)
        agg2 = _spmm(table2, src_p, dst_p, w_p)[:, :N, :]
        E_l, nrm = _dense(
            agg2, agg2, E_l,
            Wf[i], bf[i].reshape(1, D), Wb[i], bb[i].reshape(1, D),
        )
        outs.append(nrm)

    all_emb = jnp.concatenate(outs, axis=1)
    return all_emb[:N_USER], all_emb[N_USER:]
